# baseline (speedup 1.0000x reference)
import jax
import jax.numpy as jnp
from jax import lax
from jax.experimental import pallas as pl
from jax.experimental.pallas import tpu as pltpu
from jax.experimental.pallas import tpu_sc as plsc

N, P2, W2, C_KV, TOPK = 4, 64, 64, 256, 8
ROWS = N * P2 * TOPK
REGIONS = N * P2
RPB = P2 * TOPK
NW = 32
RGW = P2 * N // NW
LANES = 16
CAP = RPB


NOB = 2


def _sc_body(ridx_h, w_h, table_h, out_h,
             idx_b, w_b, mrows, oidx, rg0, rg1, ob0, ob1,
             sr0, sr1, so0, so1):
    wid = lax.axis_index("s") * 2 + lax.axis_index("c")
    batch = wid // (NW // N)
    g8 = wid % (NW // N)
    first_local = g8 * RGW
    regbase = batch * P2 + first_local

    regb, obufs = (rg0, rg1), (ob0, ob1)
    srs, sos = (sr0, sr1), (so0, so1)

    pltpu.sync_copy(ridx_h.at[pl.ds(pl.multiple_of(batch * RPB, RPB), RPB)], idx_b)
    pltpu.sync_copy(w_h.at[pl.ds(pl.multiple_of(batch * RPB, RPB), RPB)], w_b)

    def start_reg(r, slot):
        pltpu.async_copy(table_h.at[pl.ds(regbase + r, 1)], regb[slot], srs[slot])

    def wait_reg(slot):
        pltpu.make_async_copy(table_h.at[pl.ds(0, 1)], regb[slot], srs[slot]).wait()

    def wait_out(slot):
        pltpu.make_async_copy(obufs[slot], out_h.at[pl.ds(0, 1)], sos[slot]).wait()

    start_reg(0, 0)
    start_reg(1, 1)

    iota = lax.iota(jnp.int32, LANES)
    lane0 = iota == 0
    zero16 = jnp.full((LANES,), 0, jnp.int32)

    counts = []
    for R0 in range(0, RGW, 2):
        def prep_body(v, cnts):
            c0, c1 = cnts
            ids = idx_b[pl.ds(pl.multiple_of(v * LANES, LANES), LANES)]
            tvec = iota + v * LANES
            m0 = ids == jnp.int32(first_local + R0)
            m1 = ids == jnp.int32(first_local + R0 + 1)
            pos0 = c0 + plsc.cumsum(jnp.where(m0, 1, 0)) - 1
            pos1 = c1 + plsc.cumsum(jnp.where(m1, 1, 0)) - 1
            plsc.store_scatter(mrows, [jnp.int32(R0 * CAP) + pos0], tvec, mask=m0)
            plsc.store_scatter(mrows, [jnp.int32((R0 + 1) * CAP) + pos1], tvec,
                               mask=m1)
            return (c0 + plsc.all_reduce_population_count(m0),
                    c1 + plsc.all_reduce_population_count(m1))
        c0, c1 = lax.fori_loop(0, RPB // LANES, prep_body, (zero16, zero16))
        counts.append(jnp.max(c0))
        counts.append(jnp.max(c1))

    uses = tuple(jnp.int32(0) for _ in range(NOB))

    for R in range(RGW):
        wait_reg(R % 2)
        reg = regb[R % 2]
        cnt_r = counts[R]

        def make_group_body(R, reg, cnt_r):
            def group_body(jg, u):
                u = list(u)
                for k in range(NOB):
                    jj = jg * NOB + k
                    valid = jj < cnt_r
                    uk = u[k]

                    @pl.when(valid)
                    def _():
                        @pl.when(uk > 0)
                        def _():
                            wait_out(k)
                        t = plsc.load_gather(
                            mrows, [zero16 + (jnp.int32(R * CAP) + jj)])
                        wv = plsc.load_gather(w_b, [t])
                        row = t + jnp.int32(batch * RPB)
                        plsc.store_scatter(oidx, [zero16 + k, zero16],
                                           row, mask=lane0)
                        oub = obufs[k]

                        @plsc.parallel_loop(0, W2, unroll=4)
                        def _(r):
                            for h in range(C_KV // LANES):
                                oub[0, r, pl.ds(h * LANES, LANES)] = (
                                    reg[0, r, pl.ds(h * LANES, LANES)] * wv)

                        pltpu.async_copy(oub, out_h.at[oidx.at[k]], sos[k])

                    u[k] = u[k] + jnp.where(valid, 1, 0).astype(jnp.int32)
                return tuple(u)
            return group_body

        n_groups = (cnt_r + (NOB - 1)) // NOB
        uses = lax.fori_loop(0, n_groups, make_group_body(R, reg, cnt_r), uses)
        if R + 2 < RGW:
            start_reg(R + 2, R % 2)

    for k in range(NOB):
        @pl.when(uses[k] > 0)
        def _():
            wait_out(k)


@jax.jit
def _sc_gather(ridx_flat, w_flat, table):
    mesh = plsc.VectorSubcoreMesh(core_axis_name="c", subcore_axis_name="s")
    k = pl.kernel(
        _sc_body,
        out_type=jax.ShapeDtypeStruct((ROWS, W2, C_KV), jnp.float32),
        mesh=mesh,
        scratch_types=[
            pltpu.VMEM((RPB,), jnp.int32),
            pltpu.VMEM((RPB,), jnp.float32),
            pltpu.VMEM((RGW * CAP,), jnp.int32),
            pltpu.VMEM((NOB, 1), jnp.int32),
            pltpu.VMEM((1, W2, C_KV), jnp.float32),
            pltpu.VMEM((1, W2, C_KV), jnp.float32),
            pltpu.VMEM((1, W2, C_KV), jnp.float32),
            pltpu.VMEM((1, W2, C_KV), jnp.float32),
            pltpu.SemaphoreType.DMA,
            pltpu.SemaphoreType.DMA,
            pltpu.SemaphoreType.DMA,
            pltpu.SemaphoreType.DMA,
        ],
        compiler_params=pltpu.CompilerParams(
            needs_layout_passes=False,
            use_tc_tiling_on_sc=True,
        ),
    )
    return k(ridx_flat, w_flat, table)


def kernel(r_idx, r_weight, kv):
    ridx_flat = r_idx.reshape(ROWS)
    w_flat = r_weight.reshape(ROWS)
    table = kv.reshape(REGIONS, W2, C_KV)
    out = _sc_gather(ridx_flat, w_flat, table)
    return out.reshape(N, P2, TOPK, W2, C_KV)


# speedup vs baseline: 1.0857x; 1.0857x over previous
import jax
import jax.numpy as jnp
from jax import lax
from jax.experimental import pallas as pl
from jax.experimental.pallas import tpu as pltpu
from jax.experimental.pallas import tpu_sc as plsc

N, P2, W2, C_KV, TOPK = 4, 64, 64, 256, 8
ROWS = N * P2 * TOPK
REGIONS = N * P2
RPB = P2 * TOPK
NW = 32
RGW = P2 * N // NW
LANES = 16
CAP = RPB


NOB = 2


def _sc_body(ridx_h, w_h, table_h, out_h,
             idx_b, w_b, mrows, oidx, rg0, rg1, ob0, ob1,
             sr0, sr1, so0, so1):
    wid = lax.axis_index("s") * 2 + lax.axis_index("c")
    batch = wid // (NW // N)
    g8 = wid % (NW // N)
    first_local = g8 * RGW
    regbase = batch * P2 + first_local

    regb, obufs = (rg0, rg1), (ob0, ob1)
    srs, sos = (sr0, sr1), (so0, so1)

    pltpu.sync_copy(ridx_h.at[pl.ds(pl.multiple_of(batch * RPB, RPB), RPB)], idx_b)
    pltpu.sync_copy(w_h.at[pl.ds(pl.multiple_of(batch * RPB, RPB), RPB)], w_b)

    def start_reg(r, slot):
        pltpu.async_copy(table_h.at[pl.ds(regbase + r, 1)], regb[slot], srs[slot])

    def wait_reg(slot):
        pltpu.make_async_copy(table_h.at[pl.ds(0, 1)], regb[slot], srs[slot]).wait()

    def wait_out(slot):
        pltpu.make_async_copy(obufs[slot], out_h.at[pl.ds(0, 1)], sos[slot]).wait()

    start_reg(0, 0)
    start_reg(1, 1)

    iota = lax.iota(jnp.int32, LANES)
    lane0 = iota == 0
    zero16 = jnp.full((LANES,), 0, jnp.int32)

    counts = []
    for R0 in range(0, RGW, 2):
        def prep_body(v, cnts):
            c0, c1 = cnts
            ids = idx_b[pl.ds(pl.multiple_of(v * LANES, LANES), LANES)]
            tvec = iota + v * LANES
            m0 = ids == jnp.int32(first_local + R0)
            m1 = ids == jnp.int32(first_local + R0 + 1)
            pos0 = c0 + plsc.cumsum(jnp.where(m0, 1, 0)) - 1
            pos1 = c1 + plsc.cumsum(jnp.where(m1, 1, 0)) - 1
            plsc.store_scatter(mrows, [jnp.int32(R0 * CAP) + pos0], tvec, mask=m0)
            plsc.store_scatter(mrows, [jnp.int32((R0 + 1) * CAP) + pos1], tvec,
                               mask=m1)
            return (c0 + plsc.all_reduce_population_count(m0),
                    c1 + plsc.all_reduce_population_count(m1))
        c0, c1 = lax.fori_loop(0, RPB // LANES, prep_body, (zero16, zero16))
        counts.append(jnp.max(c0))
        counts.append(jnp.max(c1))

    uses = tuple(jnp.int32(0) for _ in range(NOB))

    for R in range(RGW):
        wait_reg(R % 2)
        reg = regb[R % 2]
        cnt_r = counts[R]

        def make_group_body(R, reg, cnt_r):
            def group_body(jg, u):
                u = list(u)
                for k in range(NOB):
                    jj = jg * NOB + k
                    valid = jj < cnt_r
                    uk = u[k]

                    @pl.when(valid)
                    def _():
                        @pl.when(uk > 0)
                        def _():
                            wait_out(k)
                        t = plsc.load_gather(
                            mrows, [zero16 + (jnp.int32(R * CAP) + jj)])
                        wv = plsc.load_gather(w_b, [t])
                        row = t + jnp.int32(batch * RPB)
                        plsc.store_scatter(oidx, [zero16 + k, zero16],
                                           row, mask=lane0)
                        oub = obufs[k]

                        @plsc.parallel_loop(0, W2, unroll=2)
                        def _(r):
                            for h in range(C_KV // LANES):
                                oub[0, r, pl.ds(h * LANES, LANES)] = (
                                    reg[0, r, pl.ds(h * LANES, LANES)] * wv)

                        pltpu.async_copy(oub, out_h.at[oidx.at[k]], sos[k])

                    u[k] = u[k] + jnp.where(valid, 1, 0).astype(jnp.int32)
                return tuple(u)
            return group_body

        n_groups = (cnt_r + (NOB - 1)) // NOB
        uses = lax.fori_loop(0, n_groups, make_group_body(R, reg, cnt_r), uses)
        if R + 2 < RGW:
            start_reg(R + 2, R % 2)

    for k in range(NOB):
        @pl.when(uses[k] > 0)
        def _():
            wait_out(k)


@jax.jit
def _sc_gather(ridx_flat, w_flat, table):
    mesh = plsc.VectorSubcoreMesh(core_axis_name="c", subcore_axis_name="s")
    k = pl.kernel(
        _sc_body,
        out_type=jax.ShapeDtypeStruct((ROWS, W2, C_KV), jnp.float32),
        mesh=mesh,
        scratch_types=[
            pltpu.VMEM((RPB,), jnp.int32),
            pltpu.VMEM((RPB,), jnp.float32),
            pltpu.VMEM((RGW * CAP,), jnp.int32),
            pltpu.VMEM((NOB, 1), jnp.int32),
            pltpu.VMEM((1, W2, C_KV), jnp.float32),
            pltpu.VMEM((1, W2, C_KV), jnp.float32),
            pltpu.VMEM((1, W2, C_KV), jnp.float32),
            pltpu.VMEM((1, W2, C_KV), jnp.float32),
            pltpu.SemaphoreType.DMA,
            pltpu.SemaphoreType.DMA,
            pltpu.SemaphoreType.DMA,
            pltpu.SemaphoreType.DMA,
        ],
        compiler_params=pltpu.CompilerParams(
            needs_layout_passes=False,
            use_tc_tiling_on_sc=True,
        ),
    )
    return k(ridx_flat, w_flat, table)


def kernel(r_idx, r_weight, kv):
    ridx_flat = r_idx.reshape(ROWS)
    w_flat = r_weight.reshape(ROWS)
    table = kv.reshape(REGIONS, W2, C_KV)
    out = _sc_gather(ridx_flat, w_flat, table)
    return out.reshape(N, P2, TOPK, W2, C_KV)


# speedup vs baseline: 1.1129x; 1.0250x over previous
import jax
import jax.numpy as jnp
from jax import lax
from jax.experimental import pallas as pl
from jax.experimental.pallas import tpu as pltpu
from jax.experimental.pallas import tpu_sc as plsc

N, P2, W2, C_KV, TOPK = 4, 64, 64, 256, 8
ROWS = N * P2 * TOPK
REGIONS = N * P2
RPB = P2 * TOPK
NW = 32
RGW = 8
LANES = 16
CAP = RPB
NOB = 2


def _sc_body(ridx_h, w_h, table_h, out_h,
             idx_b, w_b, mrows, cnts_l, ml0, ml1, cv0, cv1, oidx,
             rg0, rg1, ob0, ob1, sh_m, sh_c, smem_q,
             sr0, sr1, so0, so1):
    core = lax.axis_index("c")
    sub = lax.axis_index("s")
    batch = core * 2 + sub // 8
    first8 = (sub // 8) * 8
    my8 = (sub % 8) * RGW

    regb, obufs = (rg0, rg1), (ob0, ob1)
    mlists, cnt16s = (ml0, ml1), (cv0, cv1)
    srs, sos = (sr0, sr1), (so0, so1)

    pltpu.sync_copy(ridx_h.at[pl.ds(pl.multiple_of(batch * RPB, RPB), RPB)], idx_b)
    pltpu.sync_copy(w_h.at[pl.ds(pl.multiple_of(batch * RPB, RPB), RPB)], w_b)

    iota = lax.iota(jnp.int32, LANES)
    lane0 = iota == 0
    zero16 = jnp.full((LANES,), 0, jnp.int32)

    for R0 in range(0, RGW, 2):
        def prep_body(v, cnts):
            c0, c1 = cnts
            ids = idx_b[pl.ds(pl.multiple_of(v * LANES, LANES), LANES)]
            tvec = iota + v * LANES
            m0 = ids == jnp.int32(my8 + R0)
            m1 = ids == jnp.int32(my8 + R0 + 1)
            pos0 = c0 + plsc.cumsum(jnp.where(m0, 1, 0)) - 1
            pos1 = c1 + plsc.cumsum(jnp.where(m1, 1, 0)) - 1
            plsc.store_scatter(mrows, [jnp.int32(R0 * CAP) + pos0], tvec, mask=m0)
            plsc.store_scatter(mrows, [jnp.int32((R0 + 1) * CAP) + pos1], tvec,
                               mask=m1)
            return (c0 + plsc.all_reduce_population_count(m0),
                    c1 + plsc.all_reduce_population_count(m1))
        c0, c1 = lax.fori_loop(0, RPB // LANES, prep_body, (zero16, zero16))
        cnts_l[pl.ds(R0 * LANES, LANES)] = c0
        cnts_l[pl.ds((R0 + 1) * LANES, LANES)] = c1

    pltpu.sync_copy(mrows, sh_m.at[sub])
    pltpu.sync_copy(cnts_l, sh_c.at[sub])

    @pl.when(sub == first8)
    def _():
        smem_q[0] = jnp.int32(0)

    plsc.subcore_barrier()

    def pull():
        return plsc.fetch_and_add(smem_q.at[0], 1, subcore_id=first8)

    def start_reg(r, slot):
        pltpu.async_copy(table_h.at[pl.ds(batch * P2 + r, 1)], regb[slot],
                         srs[slot])

    def wait_reg(slot):
        pltpu.make_async_copy(table_h.at[pl.ds(0, 1)], regb[slot],
                              srs[slot]).wait()

    def wait_out(slot):
        pltpu.make_async_copy(obufs[slot], out_h.at[pl.ds(0, 1)],
                              sos[slot]).wait()

    r_first = [None, None]
    for k in range(2):
        r_first[k] = pull()

        @pl.when(r_first[k] < jnp.int32(P2))
        def _(k=k):
            start_reg(r_first[k], k)

    def process(rk, k, u0, u1):
        rk_c = jnp.minimum(rk, jnp.int32(P2 - 1))
        owner = first8 + lax.div(rk_c, jnp.int32(RGW))
        roff = lax.rem(rk_c, jnp.int32(RGW))
        pltpu.sync_copy(sh_c.at[owner, pl.ds(roff * LANES, LANES)], cnt16s[k])
        pltpu.sync_copy(sh_m.at[owner, pl.ds(roff * CAP, CAP)], mlists[k])

        @pl.when(rk < jnp.int32(P2))
        def _():
            wait_reg(k)

        cnt_r = jnp.where(rk < jnp.int32(P2), jnp.max(cnt16s[k][...]),
                          jnp.int32(0))
        reg, mlist = regb[k], mlists[k]

        def group_body(jg, u):
            u = list(u)
            for q in range(NOB):
                jj = jg * NOB + q
                valid = jj < cnt_r
                uq = u[q]

                @pl.when(valid)
                def _():
                    @pl.when(uq > 0)
                    def _():
                        wait_out(q)
                    t = plsc.load_gather(mlist, [zero16 + jj])
                    wv = plsc.load_gather(w_b, [t])
                    row = t + jnp.int32(batch * RPB)
                    plsc.store_scatter(oidx, [zero16 + q, zero16], row,
                                       mask=lane0)
                    oub = obufs[q]

                    @plsc.parallel_loop(0, W2, unroll=2)
                    def _(r):
                        for h in range(C_KV // LANES):
                            oub[0, r, pl.ds(h * LANES, LANES)] = (
                                reg[0, r, pl.ds(h * LANES, LANES)] * wv)

                    pltpu.async_copy(oub, out_h.at[oidx.at[q]], sos[q])

                u[q] = u[q] + jnp.where(valid, 1, 0).astype(jnp.int32)
            return tuple(u)

        n_groups = (cnt_r + (NOB - 1)) // NOB
        return lax.fori_loop(0, n_groups, group_body, (u0, u1))

    def cond(carry):
        r0, r1, u0, u1 = carry
        return jnp.logical_or(r0 < jnp.int32(P2), r1 < jnp.int32(P2))

    def body(carry):
        r0, r1, u0, u1 = carry
        rs = [r0, r1]
        for k in range(2):
            u0, u1 = process(rs[k], k, u0, u1)
            rn = pull()

            @pl.when(rn < jnp.int32(P2))
            def _(k=k, rn=rn):
                start_reg(rn, k)

            rs[k] = rn
        return (rs[0], rs[1], u0, u1)

    _, _, u0, u1 = lax.while_loop(
        cond, body, (r_first[0], r_first[1], jnp.int32(0), jnp.int32(0)))

    for q, uq in ((0, u0), (1, u1)):
        @pl.when(uq > 0)
        def _(q=q):
            wait_out(q)


@jax.jit
def _sc_gather(ridx_flat, w_flat, table):
    mesh = plsc.VectorSubcoreMesh(core_axis_name="c", subcore_axis_name="s")
    k = pl.kernel(
        _sc_body,
        out_type=jax.ShapeDtypeStruct((ROWS, W2, C_KV), jnp.float32),
        mesh=mesh,
        scratch_types=[
            pltpu.VMEM((RPB,), jnp.int32),
            pltpu.VMEM((RPB,), jnp.float32),
            pltpu.VMEM((RGW * CAP,), jnp.int32),
            pltpu.VMEM((RGW * LANES,), jnp.int32),
            pltpu.VMEM((CAP,), jnp.int32),
            pltpu.VMEM((CAP,), jnp.int32),
            pltpu.VMEM((LANES,), jnp.int32),
            pltpu.VMEM((LANES,), jnp.int32),
            pltpu.VMEM((NOB, 1), jnp.int32),
            pltpu.VMEM((1, W2, C_KV), jnp.float32),
            pltpu.VMEM((1, W2, C_KV), jnp.float32),
            pltpu.VMEM((1, W2, C_KV), jnp.float32),
            pltpu.VMEM((1, W2, C_KV), jnp.float32),
            pltpu.VMEM_SHARED((16, RGW * CAP), jnp.int32),
            pltpu.VMEM_SHARED((16, RGW * LANES), jnp.int32),
            pltpu.SMEM((1,), jnp.int32),
            pltpu.SemaphoreType.DMA,
            pltpu.SemaphoreType.DMA,
            pltpu.SemaphoreType.DMA,
            pltpu.SemaphoreType.DMA,
        ],
        compiler_params=pltpu.CompilerParams(
            needs_layout_passes=False,
            use_tc_tiling_on_sc=True,
        ),
    )
    return k(ridx_flat, w_flat, table)


def kernel(r_idx, r_weight, kv):
    ridx_flat = r_idx.reshape(ROWS)
    w_flat = r_weight.reshape(ROWS)
    table = kv.reshape(REGIONS, W2, C_KV)
    out = _sc_gather(ridx_flat, w_flat, table)
    return out.reshape(N, P2, TOPK, W2, C_KV)


# speedup vs baseline: 1.1135x; 1.0005x over previous
import jax
import jax.numpy as jnp
from jax import lax
from jax.experimental import pallas as pl
from jax.experimental.pallas import tpu as pltpu
from jax.experimental.pallas import tpu_sc as plsc

N, P2, W2, C_KV, TOPK = 4, 64, 64, 256, 8
ROWS = N * P2 * TOPK
REGIONS = N * P2
RPB = P2 * TOPK
NW = 32
RGW = 8
LANES = 16
CAP = RPB
NOB = 2


def _sc_body(ridx_h, w_h, table_h, out_h,
             idx_b, w_b, mrows, cnts_l, ml0, ml1, cv0, cv1, oidx,
             rg0, rg1, ob0, ob1, sh_m, sh_c, smem_q,
             sr0, sr1, so0, so1):
    core = lax.axis_index("c")
    sub = lax.axis_index("s")
    batch = core * 2 + sub // 8
    first8 = (sub // 8) * 8
    my8 = (sub % 8) * RGW

    regb, obufs = (rg0, rg1), (ob0, ob1)
    mlists, cnt16s = (ml0, ml1), (cv0, cv1)
    srs, sos = (sr0, sr1), (so0, so1)

    pltpu.sync_copy(ridx_h.at[pl.ds(pl.multiple_of(batch * RPB, RPB), RPB)], idx_b)
    pltpu.sync_copy(w_h.at[pl.ds(pl.multiple_of(batch * RPB, RPB), RPB)], w_b)

    iota = lax.iota(jnp.int32, LANES)
    lane0 = iota == 0
    zero16 = jnp.full((LANES,), 0, jnp.int32)

    for R0 in range(0, RGW, 2):
        def prep_body(v, cnts):
            c0, c1 = cnts
            ids = idx_b[pl.ds(pl.multiple_of(v * LANES, LANES), LANES)]
            tvec = iota + v * LANES
            m0 = ids == jnp.int32(my8 + R0)
            m1 = ids == jnp.int32(my8 + R0 + 1)
            pos0 = c0 + plsc.cumsum(jnp.where(m0, 1, 0)) - 1
            pos1 = c1 + plsc.cumsum(jnp.where(m1, 1, 0)) - 1
            plsc.store_scatter(mrows, [jnp.int32(R0 * CAP) + pos0], tvec, mask=m0)
            plsc.store_scatter(mrows, [jnp.int32((R0 + 1) * CAP) + pos1], tvec,
                               mask=m1)
            return (c0 + plsc.all_reduce_population_count(m0),
                    c1 + plsc.all_reduce_population_count(m1))
        c0, c1 = lax.fori_loop(0, RPB // LANES, prep_body, (zero16, zero16))
        cnts_l[pl.ds(R0 * LANES, LANES)] = c0
        cnts_l[pl.ds((R0 + 1) * LANES, LANES)] = c1

    pltpu.sync_copy(mrows, sh_m.at[sub])
    pltpu.sync_copy(cnts_l, sh_c.at[sub])

    @pl.when(sub == first8)
    def _():
        smem_q[0] = jnp.int32(0)

    plsc.subcore_barrier()

    def pull():
        return plsc.fetch_and_add(smem_q.at[0], 1, subcore_id=first8)

    def start_reg(r, slot):
        pltpu.async_copy(table_h.at[pl.ds(batch * P2 + r, 1)], regb[slot],
                         srs[slot])

    def wait_reg(slot):
        pltpu.make_async_copy(table_h.at[pl.ds(0, 1)], regb[slot],
                              srs[slot]).wait()

    def wait_out(slot):
        pltpu.make_async_copy(obufs[slot], out_h.at[pl.ds(0, 1)],
                              sos[slot]).wait()

    r_first = [None, None]
    for k in range(2):
        r_first[k] = pull()

        @pl.when(r_first[k] < jnp.int32(P2))
        def _(k=k):
            start_reg(r_first[k], k)

    def process(rk, k, u0, u1):
        rk_c = jnp.minimum(rk, jnp.int32(P2 - 1))
        owner = first8 + lax.div(rk_c, jnp.int32(RGW))
        roff = lax.rem(rk_c, jnp.int32(RGW))
        pltpu.sync_copy(sh_c.at[owner, pl.ds(roff * LANES, LANES)], cnt16s[k])
        pltpu.sync_copy(sh_m.at[owner, pl.ds(roff * CAP, CAP)], mlists[k])

        @pl.when(rk < jnp.int32(P2))
        def _():
            wait_reg(k)

        cnt_r = jnp.where(rk < jnp.int32(P2), jnp.max(cnt16s[k][...]),
                          jnp.int32(0))
        reg, mlist = regb[k], mlists[k]

        def group_body(jg, u):
            u = list(u)
            for q in range(NOB):
                jj = jg * NOB + q
                valid = jj < cnt_r
                uq = u[q]

                @pl.when(valid)
                def _():
                    @pl.when(uq > 0)
                    def _():
                        wait_out(q)
                    t = plsc.load_gather(mlist, [zero16 + jj])
                    wv = plsc.load_gather(w_b, [t])
                    row = t + jnp.int32(batch * RPB)
                    plsc.store_scatter(oidx, [zero16 + q, zero16], row,
                                       mask=lane0)
                    oub = obufs[q]

                    @plsc.parallel_loop(0, W2, unroll=1)
                    def _(r):
                        for h in range(C_KV // LANES):
                            oub[0, r, pl.ds(h * LANES, LANES)] = (
                                reg[0, r, pl.ds(h * LANES, LANES)] * wv)

                    pltpu.async_copy(oub, out_h.at[oidx.at[q]], sos[q])

                u[q] = u[q] + jnp.where(valid, 1, 0).astype(jnp.int32)
            return tuple(u)

        n_groups = (cnt_r + (NOB - 1)) // NOB
        return lax.fori_loop(0, n_groups, group_body, (u0, u1))

    def cond(carry):
        r0, r1, u0, u1 = carry
        return jnp.logical_or(r0 < jnp.int32(P2), r1 < jnp.int32(P2))

    def body(carry):
        r0, r1, u0, u1 = carry
        rs = [r0, r1]
        for k in range(2):
            u0, u1 = process(rs[k], k, u0, u1)
            rn = pull()

            @pl.when(rn < jnp.int32(P2))
            def _(k=k, rn=rn):
                start_reg(rn, k)

            rs[k] = rn
        return (rs[0], rs[1], u0, u1)

    _, _, u0, u1 = lax.while_loop(
        cond, body, (r_first[0], r_first[1], jnp.int32(0), jnp.int32(0)))

    for q, uq in ((0, u0), (1, u1)):
        @pl.when(uq > 0)
        def _(q=q):
            wait_out(q)


@jax.jit
def _sc_gather(ridx_flat, w_flat, table):
    mesh = plsc.VectorSubcoreMesh(core_axis_name="c", subcore_axis_name="s")
    k = pl.kernel(
        _sc_body,
        out_type=jax.ShapeDtypeStruct((ROWS, W2, C_KV), jnp.float32),
        mesh=mesh,
        scratch_types=[
            pltpu.VMEM((RPB,), jnp.int32),
            pltpu.VMEM((RPB,), jnp.float32),
            pltpu.VMEM((RGW * CAP,), jnp.int32),
            pltpu.VMEM((RGW * LANES,), jnp.int32),
            pltpu.VMEM((CAP,), jnp.int32),
            pltpu.VMEM((CAP,), jnp.int32),
            pltpu.VMEM((LANES,), jnp.int32),
            pltpu.VMEM((LANES,), jnp.int32),
            pltpu.VMEM((NOB, 1), jnp.int32),
            pltpu.VMEM((1, W2, C_KV), jnp.float32),
            pltpu.VMEM((1, W2, C_KV), jnp.float32),
            pltpu.VMEM((1, W2, C_KV), jnp.float32),
            pltpu.VMEM((1, W2, C_KV), jnp.float32),
            pltpu.VMEM_SHARED((16, RGW * CAP), jnp.int32),
            pltpu.VMEM_SHARED((16, RGW * LANES), jnp.int32),
            pltpu.SMEM((1,), jnp.int32),
            pltpu.SemaphoreType.DMA,
            pltpu.SemaphoreType.DMA,
            pltpu.SemaphoreType.DMA,
            pltpu.SemaphoreType.DMA,
        ],
        compiler_params=pltpu.CompilerParams(
            needs_layout_passes=False,
            use_tc_tiling_on_sc=True,
        ),
    )
    return k(ridx_flat, w_flat, table)


def kernel(r_idx, r_weight, kv):
    ridx_flat = r_idx.reshape(ROWS)
    w_flat = r_weight.reshape(ROWS)
    table = kv.reshape(REGIONS, W2, C_KV)
    out = _sc_gather(ridx_flat, w_flat, table)
    return out.reshape(N, P2, TOPK, W2, C_KV)
